# Initial kernel scaffold; baseline (speedup 1.0000x reference)
import jax
import jax.numpy as jnp
from jax.experimental import pallas as pl


def kernel(x, w0, b0, w1, b1, w2, b2, w3, b3, w4, b4, w5, b5, w6, b6):
    raise NotImplementedError("write your pallas kernel here")


# speedup vs baseline: 3.7950x; 3.7950x over previous
import jax
import jax.numpy as jnp
from jax.experimental import pallas as pl
from jax.experimental.pallas import tpu as pltpu

_LANES = 256
_ITEMS_PER_ROW = 32
_SLOT = 64
_GROUPS = 8
_R_BLK = 1024


def _pack_weights(w0, b0, w1, b1, w2, b2, w3, b3, w4, b4, w5, b5, w6, b6):
    f32 = jnp.float32
    eye4 = jnp.eye(4, dtype=f32)
    eye8 = jnp.eye(8, dtype=f32)

    l0 = jnp.zeros((8, 64), f32).at[:4, :32].set(w0).at[4:, 32:].set(w2)
    l1 = jnp.zeros((64, 64), f32).at[:32, :32].set(w1).at[32:, 32:].set(w3)
    l2 = jnp.zeros((64, 64), f32).at[:, :32].set(w4)
    l3 = jnp.zeros((64, 64), f32).at[:32, :32].set(w5)
    l4 = jnp.zeros((64, 8), f32).at[:32, :].set(w6)

    w1e = jnp.kron(eye4, l1)
    w2e = jnp.kron(eye4, l2)
    w3e = jnp.kron(eye4, l3)

    k0 = jnp.kron(eye4, l0)
    w0e = (jnp.kron(eye8, k0)
           .reshape(_LANES, _GROUPS, _LANES)
           .transpose(1, 0, 2))

    k4 = jnp.kron(eye4, l4)
    w4e = jnp.kron(eye8, k4).reshape(_GROUPS, _LANES, _LANES)

    ws = jnp.concatenate(
        [w0e, w1e[None], w2e[None], w3e[None], w4e], axis=0)

    b0e = jnp.tile(jnp.concatenate([b0, b2]), 4)
    b1e = jnp.tile(jnp.concatenate([b1, b3]), 4)
    b2e = jnp.tile(jnp.concatenate([b4, jnp.zeros((32,), f32)]), 4)
    b3e = jnp.tile(jnp.concatenate([b5, jnp.zeros((32,), f32)]), 4)
    b4e = jnp.tile(b6, _ITEMS_PER_ROW)
    bs = jnp.stack([b0e, b1e, b2e, b3e, b4e])[:, None, :]
    return ws, bs


def _mlp_kernel(x_ref, w_ref, b_ref, out_ref):
    x = x_ref[...]
    b0 = b_ref[0]
    b1 = b_ref[1]
    b2 = b_ref[2]
    b3 = b_ref[3]
    acc = b_ref[4] + jnp.zeros_like(x)
    for j in range(_GROUPS):
        h = jnp.dot(x, w_ref[j], preferred_element_type=jnp.float32)
        h = jnp.maximum(h + b0, 0.0)
        h = jnp.dot(h, w_ref[8], preferred_element_type=jnp.float32)
        h = jnp.maximum(h + b1, 0.0)
        h = jnp.dot(h, w_ref[9], preferred_element_type=jnp.float32)
        h = jnp.maximum(h + b2, 0.0)
        h = jnp.dot(h, w_ref[10], preferred_element_type=jnp.float32)
        h = jnp.maximum(h + b3, 0.0)
        acc = acc + jnp.dot(h, w_ref[11 + j], preferred_element_type=jnp.float32)
    out_ref[...] = acc


def kernel(x, w0, b0, w1, b1, w2, b2, w3, b3, w4, b4, w5, b5, w6, b6):
    B, D = x.shape
    ws, bs = _pack_weights(w0, b0, w1, b1, w2, b2, w3, b3, w4, b4, w5, b5,
                           w6, b6)

    items_per_blk = _ITEMS_PER_ROW * _R_BLK
    b_pad = ((B + items_per_blk - 1) // items_per_blk) * items_per_blk
    if b_pad != B:
        x = jnp.zeros((b_pad, D), x.dtype).at[:B].set(x)
    r_total = b_pad // _ITEMS_PER_ROW
    xp = x.reshape(r_total, _LANES)

    grid = (r_total // _R_BLK,)
    cost = pl.CostEstimate(
        flops=2 * 40 * r_total * _LANES * _LANES,
        transcendentals=0,
        bytes_accessed=4 * (2 * r_total * _LANES + ws.size + bs.size),
    )
    out = pl.pallas_call(
        _mlp_kernel,
        out_shape=jax.ShapeDtypeStruct((r_total, _LANES), jnp.float32),
        grid=grid,
        in_specs=[
            pl.BlockSpec((_R_BLK, _LANES), lambda i: (i, 0)),
            pl.BlockSpec((19, _LANES, _LANES), lambda i: (0, 0, 0)),
            pl.BlockSpec((5, 1, _LANES), lambda i: (0, 0, 0)),
        ],
        out_specs=pl.BlockSpec((_R_BLK, _LANES), lambda i: (i, 0)),
        compiler_params=pltpu.CompilerParams(
            dimension_semantics=("parallel",),
        ),
        cost_estimate=cost,
    )(xp, ws, bs)

    return out.reshape(b_pad, 8)[:B]


# speedup vs baseline: 3.9284x; 1.0351x over previous
import jax
import jax.numpy as jnp
from jax.experimental import pallas as pl
from jax.experimental.pallas import tpu as pltpu

_LANES = 256
_SLOTS = 4
_B_BLK = 16384
_Q = _B_BLK // _SLOTS


def _pack_weights(w0, b0, w1, b1, w2, b2, w3, b3, w4, b4, w5, b5, w6, b6):
    f32 = jnp.float32
    eye4 = jnp.eye(4, dtype=f32)

    l0 = jnp.zeros((8, 64), f32).at[:4, :32].set(w0).at[4:, 32:].set(w2)
    l1 = jnp.zeros((64, 64), f32).at[:32, :32].set(w1).at[32:, 32:].set(w3)
    l2 = jnp.zeros((64, 64), f32).at[:, :32].set(w4)
    l3 = jnp.zeros((64, 64), f32).at[:32, :32].set(w5)
    l4 = jnp.zeros((64, 8), f32).at[:32, :].set(w6)

    w0s = jnp.kron(eye4, l0).reshape(_SLOTS, 8, _LANES)
    wm = jnp.stack([jnp.kron(eye4, l1), jnp.kron(eye4, l2),
                    jnp.kron(eye4, l3)])
    w4a = jnp.kron(eye4, l4)

    b0e = jnp.tile(jnp.concatenate([b0, b2]), _SLOTS)
    b1e = jnp.tile(jnp.concatenate([b1, b3]), _SLOTS)
    b2e = jnp.tile(jnp.concatenate([b4, jnp.zeros((32,), f32)]), _SLOTS)
    b3e = jnp.tile(jnp.concatenate([b5, jnp.zeros((32,), f32)]), _SLOTS)
    bs = jnp.stack([b0e, b1e, b2e, b3e])[:, None, :]
    bo = jnp.tile(b6, _SLOTS)[None, :]
    return w0s, wm, w4a, bs, bo


def _mlp_kernel(x_ref, w0_ref, wm_ref, w4_ref, b_ref, bo_ref, out_ref):
    h = jnp.dot(x_ref[0:_Q], w0_ref[0], preferred_element_type=jnp.float32)
    for s in range(1, _SLOTS):
        h = h + jnp.dot(x_ref[s * _Q:(s + 1) * _Q], w0_ref[s],
                        preferred_element_type=jnp.float32)
    h = jnp.maximum(h + b_ref[0], 0.0)
    for l in range(3):
        h = jnp.maximum(
            jnp.dot(h, wm_ref[l], preferred_element_type=jnp.float32)
            + b_ref[l + 1], 0.0)
    o = jnp.dot(h, w4_ref[...], preferred_element_type=jnp.float32) + bo_ref[...]
    for s in range(_SLOTS):
        out_ref[s * _Q:(s + 1) * _Q] = o[:, 8 * s:8 * s + 8]


def kernel(x, w0, b0, w1, b1, w2, b2, w3, b3, w4, b4, w5, b5, w6, b6):
    B, D = x.shape
    w0s, wm, w4a, bs, bo = _pack_weights(
        w0, b0, w1, b1, w2, b2, w3, b3, w4, b4, w5, b5, w6, b6)

    b_pad = ((B + _B_BLK - 1) // _B_BLK) * _B_BLK
    if b_pad != B:
        x = jnp.zeros((b_pad, D), x.dtype).at[:B].set(x)

    grid = (b_pad // _B_BLK,)
    cost = pl.CostEstimate(
        flops=2 * 52 * b_pad * 1024,
        transcendentals=0,
        bytes_accessed=4 * (2 * b_pad * 128),
    )
    out = pl.pallas_call(
        _mlp_kernel,
        out_shape=jax.ShapeDtypeStruct((b_pad, 8), jnp.float32),
        grid=grid,
        in_specs=[
            pl.BlockSpec((_B_BLK, 8), lambda i: (i, 0)),
            pl.BlockSpec((_SLOTS, 8, _LANES), lambda i: (0, 0, 0)),
            pl.BlockSpec((3, _LANES, _LANES), lambda i: (0, 0, 0)),
            pl.BlockSpec((_LANES, 32), lambda i: (0, 0)),
            pl.BlockSpec((_SLOTS, 1, _LANES), lambda i: (0, 0, 0)),
            pl.BlockSpec((1, 32), lambda i: (0, 0)),
        ],
        out_specs=pl.BlockSpec((_B_BLK, 8), lambda i: (i, 0)),
        compiler_params=pltpu.CompilerParams(
            dimension_semantics=("parallel",),
        ),
        cost_estimate=cost,
    )(x, w0s, wm, w4a, bs, bo)

    return out[:B]


# speedup vs baseline: 23.5808x; 6.0027x over previous
import jax
import jax.numpy as jnp
from jax.experimental import pallas as pl
from jax.experimental.pallas import tpu as pltpu

_N_BLK = 16384


def _pack_weights(w0, b0, w1, b1, w2, b2, w3, b3, w4, b4, w5, b5, w6, b6):
    f32 = jnp.float32

    a0 = (jnp.zeros((65, 9), f32)
          .at[:32, :4].set(w0.T).at[32:64, 4:8].set(w2.T)
          .at[:64, 8].set(jnp.concatenate([b0, b2]))
          .at[64, 8].set(1.0))
    a1 = (jnp.zeros((65, 65), f32)
          .at[:32, :32].set(w1.T).at[32:64, 32:64].set(w3.T)
          .at[:64, 64].set(jnp.concatenate([b1, b3]))
          .at[64, 64].set(1.0))
    a2 = (jnp.zeros((33, 65), f32)
          .at[:32, :64].set(w4.T).at[:32, 64].set(b4).at[32, 64].set(1.0))
    a3 = (jnp.zeros((33, 33), f32)
          .at[:32, :32].set(w5.T).at[:32, 32].set(b5).at[32, 32].set(1.0))
    a4 = jnp.zeros((8, 33), f32).at[:, :32].set(w6.T).at[:, 32].set(b6)
    return a0, a1, a2, a3, a4


def _mlp_kernel(x_ref, a0_ref, a1_ref, a2_ref, a3_ref, a4_ref, out_ref):
    ones = jnp.ones((1, _N_BLK), jnp.float32)
    h = jnp.concatenate([x_ref[...], ones], axis=0)
    h = jnp.maximum(
        jnp.dot(a0_ref[...], h, preferred_element_type=jnp.float32), 0.0)
    h = jnp.maximum(
        jnp.dot(a1_ref[...], h, preferred_element_type=jnp.float32), 0.0)
    h = jnp.maximum(
        jnp.dot(a2_ref[...], h, preferred_element_type=jnp.float32), 0.0)
    h = jnp.maximum(
        jnp.dot(a3_ref[...], h, preferred_element_type=jnp.float32), 0.0)
    out_ref[...] = jnp.dot(a4_ref[...], h,
                           preferred_element_type=jnp.float32)


def kernel(x, w0, b0, w1, b1, w2, b2, w3, b3, w4, b4, w5, b5, w6, b6):
    B, D = x.shape
    aws = _pack_weights(w0, b0, w1, b1, w2, b2, w3, b3, w4, b4, w5, b5,
                        w6, b6)

    xt = x.T
    b_pad = ((B + _N_BLK - 1) // _N_BLK) * _N_BLK
    if b_pad != B:
        xt = jnp.zeros((D, b_pad), xt.dtype).at[:, :B].set(xt)

    grid = (b_pad // _N_BLK,)
    cost = pl.CostEstimate(
        flops=2 * 8000 * b_pad,
        transcendentals=0,
        bytes_accessed=4 * 16 * b_pad,
    )
    out = pl.pallas_call(
        _mlp_kernel,
        out_shape=jax.ShapeDtypeStruct((8, b_pad), jnp.float32),
        grid=grid,
        in_specs=[
            pl.BlockSpec((8, _N_BLK), lambda i: (0, i)),
            pl.BlockSpec((65, 9), lambda i: (0, 0)),
            pl.BlockSpec((65, 65), lambda i: (0, 0)),
            pl.BlockSpec((33, 65), lambda i: (0, 0)),
            pl.BlockSpec((33, 33), lambda i: (0, 0)),
            pl.BlockSpec((8, 33), lambda i: (0, 0)),
        ],
        out_specs=pl.BlockSpec((8, _N_BLK), lambda i: (0, i)),
        compiler_params=pltpu.CompilerParams(
            dimension_semantics=("parallel",),
        ),
        cost_estimate=cost,
    )(xt, *aws)

    return out[:, :B].T


# speedup vs baseline: 29.9366x; 1.2695x over previous
import jax
import jax.numpy as jnp
from jax.experimental import pallas as pl
from jax.experimental.pallas import tpu as pltpu

_N_BLK = 32768


def _pack_weights(w0, b0, w1, b1, w2, b2, w3, b3, w4, b4, w5, b5, w6, b6):
    f32 = jnp.float32
    bf16 = jnp.bfloat16

    a0 = (jnp.zeros((64, 8), f32)
          .at[:32, :4].set(w0.T).at[32:, 4:].set(w2.T)).astype(bf16)
    a1 = (jnp.zeros((64, 64), f32)
          .at[:32, :32].set(w1.T).at[32:, 32:].set(w3.T)).astype(bf16)
    a2 = w4.T.astype(bf16)
    a3 = w5.T.astype(bf16)
    a4 = w6.T.astype(bf16)
    c0 = jnp.concatenate([b0, b2])[:, None]
    c1 = jnp.concatenate([b1, b3])[:, None]
    return a0, a1, a2, a3, a4, c0, c1, b4[:, None], b5[:, None], b6[:, None]


def _mlp_kernel(x_ref, a0_ref, a1_ref, a2_ref, a3_ref, a4_ref,
                c0_ref, c1_ref, c2_ref, c3_ref, c4_ref, out_ref):
    bf16 = jnp.bfloat16
    h = x_ref[...].astype(bf16)
    for a_ref, c_ref in ((a0_ref, c0_ref), (a1_ref, c1_ref),
                         (a2_ref, c2_ref), (a3_ref, c3_ref)):
        z = jnp.dot(a_ref[...], h, preferred_element_type=jnp.float32)
        h = jnp.maximum((z + c_ref[...]).astype(bf16), 0)
    out_ref[...] = (
        jnp.dot(a4_ref[...], h, preferred_element_type=jnp.float32)
        + c4_ref[...])


def kernel(x, w0, b0, w1, b1, w2, b2, w3, b3, w4, b4, w5, b5, w6, b6):
    B, D = x.shape
    packed = _pack_weights(w0, b0, w1, b1, w2, b2, w3, b3, w4, b4, w5, b5,
                           w6, b6)

    xt = x.T
    b_pad = ((B + _N_BLK - 1) // _N_BLK) * _N_BLK
    if b_pad != B:
        xt = jnp.zeros((D, b_pad), xt.dtype).at[:, :B].set(xt)

    grid = (b_pad // _N_BLK,)
    cost = pl.CostEstimate(
        flops=2 * 8000 * b_pad,
        transcendentals=0,
        bytes_accessed=4 * 16 * b_pad,
    )
    wspecs = [pl.BlockSpec(w.shape, lambda i: (0, 0)) for w in packed]
    out = pl.pallas_call(
        _mlp_kernel,
        out_shape=jax.ShapeDtypeStruct((8, b_pad), jnp.float32),
        grid=grid,
        in_specs=[pl.BlockSpec((8, _N_BLK), lambda i: (0, i))] + wspecs,
        out_specs=pl.BlockSpec((8, _N_BLK), lambda i: (0, i)),
        compiler_params=pltpu.CompilerParams(
            dimension_semantics=("parallel",),
        ),
        cost_estimate=cost,
    )(xt, *packed)

    return out[:, :B].T


# speedup vs baseline: 30.4874x; 1.0184x over previous
import jax
import jax.numpy as jnp
from jax.experimental import pallas as pl
from jax.experimental.pallas import tpu as pltpu

_N_BLK = 65536


def _pack_weights(w0, b0, w1, b1, w2, b2, w3, b3, w4, b4, w5, b5, w6, b6):
    f32 = jnp.float32
    bf16 = jnp.bfloat16

    a0 = (jnp.zeros((64, 8), f32)
          .at[:32, :4].set(w0.T).at[32:, 4:].set(w2.T)).astype(bf16)
    a1 = (jnp.zeros((64, 64), f32)
          .at[:32, :32].set(w1.T).at[32:, 32:].set(w3.T)).astype(bf16)
    a2 = w4.T.astype(bf16)
    a3 = w5.T.astype(bf16)
    a4 = w6.T.astype(bf16)
    c0 = jnp.concatenate([b0, b2])[:, None]
    c1 = jnp.concatenate([b1, b3])[:, None]
    return a0, a1, a2, a3, a4, c0, c1, b4[:, None], b5[:, None], b6[:, None]


def _mlp_kernel(x_ref, a0_ref, a1_ref, a2_ref, a3_ref, a4_ref,
                c0_ref, c1_ref, c2_ref, c3_ref, c4_ref, out_ref):
    bf16 = jnp.bfloat16
    h = x_ref[...].astype(bf16)
    for a_ref, c_ref in ((a0_ref, c0_ref), (a1_ref, c1_ref),
                         (a2_ref, c2_ref), (a3_ref, c3_ref)):
        z = jnp.dot(a_ref[...], h, preferred_element_type=jnp.float32)
        h = jnp.maximum((z + c_ref[...]).astype(bf16), 0)
    out_ref[...] = (
        jnp.dot(a4_ref[...], h, preferred_element_type=jnp.float32)
        + c4_ref[...])


def kernel(x, w0, b0, w1, b1, w2, b2, w3, b3, w4, b4, w5, b5, w6, b6):
    B, D = x.shape
    packed = _pack_weights(w0, b0, w1, b1, w2, b2, w3, b3, w4, b4, w5, b5,
                           w6, b6)

    xt = x.T
    b_pad = ((B + _N_BLK - 1) // _N_BLK) * _N_BLK
    if b_pad != B:
        xt = jnp.zeros((D, b_pad), xt.dtype).at[:, :B].set(xt)

    grid = (b_pad // _N_BLK,)
    cost = pl.CostEstimate(
        flops=2 * 8000 * b_pad,
        transcendentals=0,
        bytes_accessed=4 * 16 * b_pad,
    )
    wspecs = [pl.BlockSpec(w.shape, lambda i: (0, 0)) for w in packed]
    out = pl.pallas_call(
        _mlp_kernel,
        out_shape=jax.ShapeDtypeStruct((8, b_pad), jnp.float32),
        grid=grid,
        in_specs=[pl.BlockSpec((8, _N_BLK), lambda i: (0, i))] + wspecs,
        out_specs=pl.BlockSpec((8, _N_BLK), lambda i: (0, i)),
        compiler_params=pltpu.CompilerParams(
            dimension_semantics=("parallel",),
        ),
        cost_estimate=cost,
    )(xt, *packed)

    return out[:, :B].T


# speedup vs baseline: 32.0000x; 1.0496x over previous
import jax
import jax.numpy as jnp
from jax.experimental import pallas as pl
from jax.experimental.pallas import tpu as pltpu

_N_BLK = 65536


def _pack_weights(w0, b0, w1, b1, w2, b2, w3, b3, w4, b4, w5, b5, w6, b6):
    f32 = jnp.float32
    bf16 = jnp.bfloat16

    a0 = (jnp.zeros((64, 8), f32)
          .at[:32, :4].set(w0.T).at[32:, 4:].set(w2.T)).astype(bf16)
    a1 = (jnp.zeros((64, 64), f32)
          .at[:32, :32].set(w1.T).at[32:, 32:].set(w3.T)).astype(bf16)
    a2 = w4.T.astype(bf16)
    a3 = w5.T.astype(bf16)
    a4 = w6.T.astype(bf16)
    c0 = jnp.concatenate([b0, b2])[:, None]
    c1 = jnp.concatenate([b1, b3])[:, None]
    return a0, a1, a2, a3, a4, c0, c1, b4[:, None], b5[:, None], b6[:, None]


def _mlp_kernel(x_ref, a0_ref, a1_ref, a2_ref, a3_ref, a4_ref,
                c0_ref, c1_ref, c2_ref, c3_ref, c4_ref, out_ref):
    bf16 = jnp.bfloat16
    h = x_ref[...].astype(bf16)
    for a_ref, c_ref in ((a0_ref, c0_ref), (a1_ref, c1_ref),
                         (a2_ref, c2_ref), (a3_ref, c3_ref)):
        z = jnp.dot(a_ref[...], h, preferred_element_type=jnp.float32)
        h = jnp.maximum((z + c_ref[...]).astype(bf16), 0)
    out_ref[...] = (
        jnp.dot(a4_ref[...], h, preferred_element_type=jnp.float32)
        + c4_ref[...])


def kernel(x, w0, b0, w1, b1, w2, b2, w3, b3, w4, b4, w5, b5, w6, b6):
    B, D = x.shape
    packed = _pack_weights(w0, b0, w1, b1, w2, b2, w3, b3, w4, b4, w5, b5,
                           w6, b6)

    xt = x.T
    b_pad = ((B + _N_BLK - 1) // _N_BLK) * _N_BLK
    if b_pad != B:
        xt = jnp.zeros((D, b_pad), xt.dtype).at[:, :B].set(xt)

    grid = (b_pad // _N_BLK,)
    cost = pl.CostEstimate(
        flops=2 * 8000 * b_pad,
        transcendentals=0,
        bytes_accessed=4 * 16 * b_pad,
    )
    wspecs = [pl.BlockSpec(w.shape, lambda i: (0, 0)) for w in packed]
    out = pl.pallas_call(
        _mlp_kernel,
        out_shape=jax.ShapeDtypeStruct((8, b_pad), jnp.float32),
        grid=grid,
        in_specs=[pl.BlockSpec((8, _N_BLK), lambda i: (0, i))] + wspecs,
        out_specs=pl.BlockSpec((8, _N_BLK), lambda i: (0, i)),
        compiler_params=pltpu.CompilerParams(
            dimension_semantics=("parallel",),
            allow_input_fusion=[False] + [True] * len(packed),
        ),
        cost_estimate=cost,
    )(xt, *packed)

    return out[:, :B].T
